# baseline (speedup 1.0000x reference)
import functools

import jax
import jax.numpy as jnp
from jax import lax
from jax.experimental import pallas as pl
from jax.experimental.pallas import tpu as pltpu
from jax.experimental.pallas import tpu_sc as plsc

NODE_CNT = 1000000
OUT_FEAT = 64
BATCH = 16384

_INFO = plsc.get_sparse_core_info()
_NC, _NS, _L = _INFO.num_cores, _INFO.num_subcores, _INFO.num_lanes
_NW = _NC * _NS
_B_PER_W = BATCH // _NW
_CHUNK = 128
_NCHUNK = _B_PER_W // _CHUNK
_GROUP = 16

_SPLIT = 512000
_TBLK = 256
_TGRID = _SPLIT // _TBLK


def _tr_body(in0_ref, in1_ref, out_ref):
    t0 = in0_ref[...].T
    t1 = in1_ref[...].T
    out_ref[...] = jnp.concatenate([t0, t1], axis=1)


def _transpose(tab_t):
    return pl.pallas_call(
        _tr_body,
        grid=(_TGRID,),
        in_specs=[
            pl.BlockSpec((OUT_FEAT, _TBLK), lambda i: (0, i)),
            pl.BlockSpec((OUT_FEAT, _TBLK),
                         lambda i: (0, jnp.minimum(i + _SPLIT // _TBLK,
                                                   NODE_CNT // _TBLK))),
        ],
        out_specs=pl.BlockSpec((_TBLK, 2 * OUT_FEAT), lambda i: (i, 0)),
        out_shape=jax.ShapeDtypeStruct((_SPLIT, 2 * OUT_FEAT), jnp.float32),
    )(tab_t, tab_t)


def _body(table_hbm, phys_hbm, par_hbm, out_hbm, phys_v, par_v, rows_v,
          out_v, sem0, sem1):
    wid = lax.axis_index("s") * _NC + lax.axis_index("c")
    base = wid * _B_PER_W
    sems = (sem0, sem1)

    pltpu.sync_copy(phys_hbm.at[pl.ds(base, _B_PER_W)], phys_v)
    pltpu.sync_copy(par_hbm.at[pl.ds(base, _B_PER_W)], par_v)

    def fire(c):
        return pltpu.async_copy(
            table_hbm.at[phys_v.at[pl.ds(c * _CHUNK, _CHUNK)]],
            rows_v.at[c % 2],
            sems[c % 2],
        )

    def compute(c):
        b = c % 2

        def group_body(g, carry):
            row0 = g * _GROUP
            pv = par_v[pl.ds(c * _CHUNK + row0, _GROUP)]
            for t in range(_GROUP):
                row = row0 + t
                w = lax.gather(
                    pv, jnp.full((_L, 1), t, jnp.int32),
                    lax.GatherDimensionNumbers(offset_dims=(),
                                               collapsed_slice_dims=(0,),
                                               start_index_map=(0,)),
                    slice_sizes=(1,),
                    mode=lax.GatherScatterMode.PROMISE_IN_BOUNDS)
                for j in range(OUT_FEAT // _L):
                    lo = rows_v[b, row, pl.ds(j * _L, _L)]
                    hi = rows_v[b, row, pl.ds(OUT_FEAT + j * _L, _L)]
                    vals = lo + w * (hi - lo)
                    out_v[c * _CHUNK + row,
                          pl.ds(j * _L, _L)] = jnp.maximum(vals, 0.0)
            return carry

        lax.fori_loop(0, _CHUNK // _GROUP, group_body, 0)

    cps = [fire(0), fire(1)]
    for c in range(_NCHUNK):
        cps[c % 2].wait()
        compute(c)
        if c + 2 < _NCHUNK:
            cps[c % 2] = fire(c + 2)

    pltpu.sync_copy(out_v, out_hbm.at[pl.ds(base, _B_PER_W)])


def kernel(nodes, table):
    idx = nodes.astype(jnp.int32)
    phys = jnp.where(idx < _SPLIT, idx, idx - _SPLIT)
    par = (idx >= _SPLIT).astype(jnp.float32)
    tab2 = _transpose(table.T)
    mesh = plsc.VectorSubcoreMesh(core_axis_name="c", subcore_axis_name="s")
    k = functools.partial(
        pl.kernel,
        mesh=mesh,
        out_type=jax.ShapeDtypeStruct((BATCH, OUT_FEAT), jnp.float32),
        scratch_types=[
            pltpu.VMEM((_B_PER_W,), jnp.int32),
            pltpu.VMEM((_B_PER_W,), jnp.float32),
            pltpu.VMEM((2, _CHUNK, 2 * OUT_FEAT), jnp.float32),
            pltpu.VMEM((_B_PER_W, OUT_FEAT), jnp.float32),
            pltpu.SemaphoreType.DMA,
            pltpu.SemaphoreType.DMA,
        ],
        compiler_params=pltpu.CompilerParams(use_tc_tiling_on_sc=True),
    )(_body)
    return k(tab2, phys, par)


# speedup vs baseline: 5.2932x; 5.2932x over previous
import functools

import jax
import jax.numpy as jnp
from jax import lax
from jax.experimental import pallas as pl
from jax.experimental.pallas import tpu as pltpu
from jax.experimental.pallas import tpu_sc as plsc

NODE_CNT = 1000000
OUT_FEAT = 64
BATCH = 16384

_INFO = plsc.get_sparse_core_info()
_NC, _NS, _L = _INFO.num_cores, _INFO.num_subcores, _INFO.num_lanes
_NW = _NC * _NS
_B_PER_W = BATCH // _NW
_CHUNK = 128
_NCHUNK = _B_PER_W // _CHUNK
_GROUP = 16

_SPLIT = 512000
_TBLK = 4096
_TGRID = _SPLIT // _TBLK


def _tr_body(in0_ref, in1_ref, out_ref):
    x = jnp.concatenate([in0_ref[...], in1_ref[...]], axis=0)
    eye = jnp.eye(2 * OUT_FEAT, dtype=jnp.float32)
    out_ref[...] = lax.dot_general(x, eye, (((0,), (0,)), ((), ())))


def _transpose(tab_t):
    return pl.pallas_call(
        _tr_body,
        grid=(_TGRID,),
        in_specs=[
            pl.BlockSpec((OUT_FEAT, _TBLK), lambda i: (0, i)),
            pl.BlockSpec((OUT_FEAT, _TBLK),
                         lambda i: (0, jnp.minimum(i + _SPLIT // _TBLK,
                                                   NODE_CNT // _TBLK))),
        ],
        out_specs=pl.BlockSpec((_TBLK, 2 * OUT_FEAT), lambda i: (i, 0)),
        out_shape=jax.ShapeDtypeStruct((_SPLIT, 2 * OUT_FEAT), jnp.float32),
    )(tab_t, tab_t)


def _body(table_hbm, phys_hbm, par_hbm, out_hbm, phys_v, par_v, rows_v,
          out_v, sem0, sem1):
    wid = lax.axis_index("s") * _NC + lax.axis_index("c")
    base = wid * _B_PER_W
    sems = (sem0, sem1)

    pltpu.sync_copy(phys_hbm.at[pl.ds(base, _B_PER_W)], phys_v)
    pltpu.sync_copy(par_hbm.at[pl.ds(base, _B_PER_W)], par_v)

    def fire(c):
        return pltpu.async_copy(
            table_hbm.at[phys_v.at[pl.ds(c * _CHUNK, _CHUNK)]],
            rows_v.at[c % 2],
            sems[c % 2],
        )

    def compute(c):
        b = c % 2

        def group_body(g, carry):
            row0 = g * _GROUP
            pv = par_v[pl.ds(c * _CHUNK + row0, _GROUP)]
            for t in range(_GROUP):
                row = row0 + t
                w = lax.gather(
                    pv, jnp.full((_L, 1), t, jnp.int32),
                    lax.GatherDimensionNumbers(offset_dims=(),
                                               collapsed_slice_dims=(0,),
                                               start_index_map=(0,)),
                    slice_sizes=(1,),
                    mode=lax.GatherScatterMode.PROMISE_IN_BOUNDS)
                for j in range(OUT_FEAT // _L):
                    lo = rows_v[b, row, pl.ds(j * _L, _L)]
                    hi = rows_v[b, row, pl.ds(OUT_FEAT + j * _L, _L)]
                    vals = lo + w * (hi - lo)
                    out_v[c * _CHUNK + row,
                          pl.ds(j * _L, _L)] = jnp.maximum(vals, 0.0)
            return carry

        lax.fori_loop(0, _CHUNK // _GROUP, group_body, 0)

    cps = [fire(0), fire(1)]
    for c in range(_NCHUNK):
        cps[c % 2].wait()
        compute(c)
        if c + 2 < _NCHUNK:
            cps[c % 2] = fire(c + 2)

    pltpu.sync_copy(out_v, out_hbm.at[pl.ds(base, _B_PER_W)])


def kernel(nodes, table):
    idx = nodes.astype(jnp.int32)
    phys = jnp.where(idx < _SPLIT, idx, idx - _SPLIT)
    par = (idx >= _SPLIT).astype(jnp.float32)
    tab2 = _transpose(table.T)
    mesh = plsc.VectorSubcoreMesh(core_axis_name="c", subcore_axis_name="s")
    k = functools.partial(
        pl.kernel,
        mesh=mesh,
        out_type=jax.ShapeDtypeStruct((BATCH, OUT_FEAT), jnp.float32),
        scratch_types=[
            pltpu.VMEM((_B_PER_W,), jnp.int32),
            pltpu.VMEM((_B_PER_W,), jnp.float32),
            pltpu.VMEM((2, _CHUNK, 2 * OUT_FEAT), jnp.float32),
            pltpu.VMEM((_B_PER_W, OUT_FEAT), jnp.float32),
            pltpu.SemaphoreType.DMA,
            pltpu.SemaphoreType.DMA,
        ],
        compiler_params=pltpu.CompilerParams(use_tc_tiling_on_sc=True),
    )(_body)
    return k(tab2, phys, par)


# speedup vs baseline: 6.1227x; 1.1567x over previous
import functools

import jax
import jax.numpy as jnp
from jax import lax
from jax.experimental import pallas as pl
from jax.experimental.pallas import tpu as pltpu
from jax.experimental.pallas import tpu_sc as plsc

NODE_CNT = 1000000
OUT_FEAT = 64
BATCH = 16384

_INFO = plsc.get_sparse_core_info()
_NC, _NS, _L = _INFO.num_cores, _INFO.num_subcores, _INFO.num_lanes
_NW = _NC * _NS
_B_PER_W = BATCH // _NW
_CHUNK = 128
_NCHUNK = _B_PER_W // _CHUNK
_GROUP = 16

_SPLIT = 512000
_TBLK = 12800
_TGRID = _SPLIT // _TBLK


def _tr_body(in0_ref, in1_ref, out_ref):
    x = jnp.concatenate([in0_ref[...], in1_ref[...]], axis=0)
    eye = jnp.eye(2 * OUT_FEAT, dtype=jnp.float32)
    out_ref[...] = lax.dot_general(x, eye, (((0,), (0,)), ((), ())))


def _transpose(tab_t):
    return pl.pallas_call(
        _tr_body,
        grid=(_TGRID,),
        in_specs=[
            pl.BlockSpec((OUT_FEAT, _TBLK), lambda i: (0, i)),
            pl.BlockSpec((OUT_FEAT, _TBLK),
                         lambda i: (0, jnp.minimum(i + _SPLIT // _TBLK,
                                                   NODE_CNT // _TBLK))),
        ],
        out_specs=pl.BlockSpec((_TBLK, 2 * OUT_FEAT), lambda i: (i, 0)),
        out_shape=jax.ShapeDtypeStruct((_SPLIT, 2 * OUT_FEAT), jnp.float32),
    )(tab_t, tab_t)


def _body(table_hbm, phys_hbm, par_hbm, out_hbm, phys_v, par_v, rows_v,
          out_v, sem0, sem1):
    wid = lax.axis_index("s") * _NC + lax.axis_index("c")
    base = wid * _B_PER_W
    sems = (sem0, sem1)

    pltpu.sync_copy(phys_hbm.at[pl.ds(base, _B_PER_W)], phys_v)
    pltpu.sync_copy(par_hbm.at[pl.ds(base, _B_PER_W)], par_v)

    def fire(c):
        return pltpu.async_copy(
            table_hbm.at[phys_v.at[pl.ds(c * _CHUNK, _CHUNK)]],
            rows_v.at[c % 2],
            sems[c % 2],
        )

    def compute(c):
        b = c % 2

        def group_body(g, carry):
            row0 = g * _GROUP
            pv = par_v[pl.ds(c * _CHUNK + row0, _GROUP)]
            for t in range(_GROUP):
                row = row0 + t
                w = lax.gather(
                    pv, jnp.full((_L, 1), t, jnp.int32),
                    lax.GatherDimensionNumbers(offset_dims=(),
                                               collapsed_slice_dims=(0,),
                                               start_index_map=(0,)),
                    slice_sizes=(1,),
                    mode=lax.GatherScatterMode.PROMISE_IN_BOUNDS)
                for j in range(OUT_FEAT // _L):
                    lo = rows_v[b, row, pl.ds(j * _L, _L)]
                    hi = rows_v[b, row, pl.ds(OUT_FEAT + j * _L, _L)]
                    vals = lo + w * (hi - lo)
                    out_v[c * _CHUNK + row,
                          pl.ds(j * _L, _L)] = jnp.maximum(vals, 0.0)
            return carry

        lax.fori_loop(0, _CHUNK // _GROUP, group_body, 0)

    cps = [fire(0), fire(1)]
    for c in range(_NCHUNK):
        cps[c % 2].wait()
        compute(c)
        if c + 2 < _NCHUNK:
            cps[c % 2] = fire(c + 2)

    pltpu.sync_copy(out_v, out_hbm.at[pl.ds(base, _B_PER_W)])


def kernel(nodes, table):
    idx = nodes.astype(jnp.int32)
    phys = jnp.where(idx < _SPLIT, idx, idx - _SPLIT)
    par = (idx >= _SPLIT).astype(jnp.float32)
    tab2 = _transpose(table.T)
    mesh = plsc.VectorSubcoreMesh(core_axis_name="c", subcore_axis_name="s")
    k = functools.partial(
        pl.kernel,
        mesh=mesh,
        out_type=jax.ShapeDtypeStruct((BATCH, OUT_FEAT), jnp.float32),
        scratch_types=[
            pltpu.VMEM((_B_PER_W,), jnp.int32),
            pltpu.VMEM((_B_PER_W,), jnp.float32),
            pltpu.VMEM((2, _CHUNK, 2 * OUT_FEAT), jnp.float32),
            pltpu.VMEM((_B_PER_W, OUT_FEAT), jnp.float32),
            pltpu.SemaphoreType.DMA,
            pltpu.SemaphoreType.DMA,
        ],
        compiler_params=pltpu.CompilerParams(use_tc_tiling_on_sc=True),
    )(_body)
    return k(tab2, phys, par)


# speedup vs baseline: 6.1821x; 1.0097x over previous
import functools

import jax
import jax.numpy as jnp
from jax import lax
from jax.experimental import pallas as pl
from jax.experimental.pallas import tpu as pltpu
from jax.experimental.pallas import tpu_sc as plsc

NODE_CNT = 1000000
OUT_FEAT = 64
BATCH = 16384

_INFO = plsc.get_sparse_core_info()
_NC, _NS, _L = _INFO.num_cores, _INFO.num_subcores, _INFO.num_lanes
_NW = _NC * _NS
_B_PER_W = BATCH // _NW
_CHUNK = 128
_NCHUNK = _B_PER_W // _CHUNK
_GROUP = 16

_SPLIT = 512000
_TBLK = 25600
_TGRID = _SPLIT // _TBLK


def _tr_body(in0_ref, in1_ref, out_ref):
    x = jnp.concatenate([in0_ref[...], in1_ref[...]], axis=0)
    eye = jnp.eye(2 * OUT_FEAT, dtype=jnp.float32)
    out_ref[...] = lax.dot_general(x, eye, (((0,), (0,)), ((), ())))


def _transpose(tab_t):
    return pl.pallas_call(
        _tr_body,
        grid=(_TGRID,),
        in_specs=[
            pl.BlockSpec((OUT_FEAT, _TBLK), lambda i: (0, i)),
            pl.BlockSpec((OUT_FEAT, _TBLK),
                         lambda i: (0, jnp.minimum(i + _SPLIT // _TBLK,
                                                   NODE_CNT // _TBLK))),
        ],
        out_specs=pl.BlockSpec((_TBLK, 2 * OUT_FEAT), lambda i: (i, 0)),
        out_shape=jax.ShapeDtypeStruct((_SPLIT, 2 * OUT_FEAT), jnp.float32),
    )(tab_t, tab_t)


def _body(table_hbm, phys_hbm, par_hbm, out_hbm, phys_v, par_v, rows_v,
          out_v, sem0, sem1):
    wid = lax.axis_index("s") * _NC + lax.axis_index("c")
    base = wid * _B_PER_W
    sems = (sem0, sem1)

    pltpu.sync_copy(phys_hbm.at[pl.ds(base, _B_PER_W)], phys_v)
    pltpu.sync_copy(par_hbm.at[pl.ds(base, _B_PER_W)], par_v)

    def fire(c):
        return pltpu.async_copy(
            table_hbm.at[phys_v.at[pl.ds(c * _CHUNK, _CHUNK)]],
            rows_v.at[c % 2],
            sems[c % 2],
        )

    def compute(c):
        b = c % 2

        def group_body(g, carry):
            row0 = g * _GROUP
            pv = par_v[pl.ds(c * _CHUNK + row0, _GROUP)]
            for t in range(_GROUP):
                row = row0 + t
                w = lax.gather(
                    pv, jnp.full((_L, 1), t, jnp.int32),
                    lax.GatherDimensionNumbers(offset_dims=(),
                                               collapsed_slice_dims=(0,),
                                               start_index_map=(0,)),
                    slice_sizes=(1,),
                    mode=lax.GatherScatterMode.PROMISE_IN_BOUNDS)
                for j in range(OUT_FEAT // _L):
                    lo = rows_v[b, row, pl.ds(j * _L, _L)]
                    hi = rows_v[b, row, pl.ds(OUT_FEAT + j * _L, _L)]
                    vals = lo + w * (hi - lo)
                    out_v[c * _CHUNK + row,
                          pl.ds(j * _L, _L)] = jnp.maximum(vals, 0.0)
            return carry

        lax.fori_loop(0, _CHUNK // _GROUP, group_body, 0)

    cps = [fire(0), fire(1)]
    for c in range(_NCHUNK):
        cps[c % 2].wait()
        compute(c)
        if c + 2 < _NCHUNK:
            cps[c % 2] = fire(c + 2)

    pltpu.sync_copy(out_v, out_hbm.at[pl.ds(base, _B_PER_W)])


def kernel(nodes, table):
    idx = nodes.astype(jnp.int32)
    phys = jnp.where(idx < _SPLIT, idx, idx - _SPLIT)
    par = (idx >= _SPLIT).astype(jnp.float32)
    tab2 = _transpose(table.T)
    mesh = plsc.VectorSubcoreMesh(core_axis_name="c", subcore_axis_name="s")
    k = functools.partial(
        pl.kernel,
        mesh=mesh,
        out_type=jax.ShapeDtypeStruct((BATCH, OUT_FEAT), jnp.float32),
        scratch_types=[
            pltpu.VMEM((_B_PER_W,), jnp.int32),
            pltpu.VMEM((_B_PER_W,), jnp.float32),
            pltpu.VMEM((2, _CHUNK, 2 * OUT_FEAT), jnp.float32),
            pltpu.VMEM((_B_PER_W, OUT_FEAT), jnp.float32),
            pltpu.SemaphoreType.DMA,
            pltpu.SemaphoreType.DMA,
        ],
        compiler_params=pltpu.CompilerParams(use_tc_tiling_on_sc=True),
    )(_body)
    return k(tab2, phys, par)
